# baseline (speedup 1.0000x reference)
import functools

import jax
import jax.numpy as jnp
import numpy as np
from jax import lax
from jax.experimental import pallas as pl
from jax.experimental.pallas import tpu as pltpu
from jax.experimental.pallas import tpu_sc as plsc

_NUM_NODES = 10000
_NUM_EDGES = 320000
_EMB_DIM = 128
_EDGE_PROBA = _NUM_EDGES / (_NUM_NODES ** 2 - _NUM_NODES)
_EPS = np.float32(-np.log(1.0 - _EDGE_PROBA))

_NW = 32
_E_PER_W = _NUM_EDGES // _NW
_CH = 80
_NCH = _E_PER_W // _CH

_mesh = plsc.VectorSubcoreMesh(core_axis_name="c", subcore_axis_name="s")


@functools.partial(
    pl.kernel,
    mesh=_mesh,
    out_type=jax.ShapeDtypeStruct((_NUM_EDGES,), jnp.float32),
    scratch_types=[
        pltpu.VMEM((_CH,), jnp.int32), pltpu.VMEM((_CH,), jnp.int32),
        pltpu.VMEM((_CH,), jnp.int32), pltpu.VMEM((_CH,), jnp.int32),
        pltpu.VMEM((_CH, _EMB_DIM), jnp.float32),
        pltpu.VMEM((_CH, _EMB_DIM), jnp.float32),
        pltpu.VMEM((_CH, _EMB_DIM), jnp.float32),
        pltpu.VMEM((_CH, _EMB_DIM), jnp.float32),
        pltpu.VMEM((_CH,), jnp.float32), pltpu.VMEM((_CH,), jnp.float32),
        pltpu.SemaphoreType.DMA, pltpu.SemaphoreType.DMA,
        pltpu.SemaphoreType.DMA, pltpu.SemaphoreType.DMA,
        pltpu.SemaphoreType.DMA, pltpu.SemaphoreType.DMA,
    ],
)
def _berpo_sc(emb_hbm, e1_hbm, e2_hbm, out_hbm,
              idx1_a, idx2_a, idx1_b, idx2_b,
              rows1_a, rows2_a, rows1_b, rows2_b,
              out_a, out_b,
              sem_idx_a, sem_idx_b, sem_g_a, sem_g_b, sem_o_a, sem_o_b):
    wid = lax.axis_index("s") * 2 + lax.axis_index("c")
    base = wid * _E_PER_W
    lanes = lax.iota(jnp.int32, 16)

    bufs = ((idx1_a, idx2_a, rows1_a, rows2_a, out_a,
             sem_idx_a, sem_g_a, sem_o_a),
            (idx1_b, idx2_b, rows1_b, rows2_b, out_b,
             sem_idx_b, sem_g_b, sem_o_b))

    def off_of(j):
        return pl.multiple_of(base + j * _CH, 8)

    def idx_cps(j, p):
        idx1_v, idx2_v, sem = bufs[p][0], bufs[p][1], bufs[p][5]
        off = off_of(j)
        return (pltpu.make_async_copy(e1_hbm.at[pl.ds(off, _CH)], idx1_v, sem),
                pltpu.make_async_copy(e2_hbm.at[pl.ds(off, _CH)], idx2_v, sem))

    def g_cps(p):
        idx1_v, idx2_v, rows1_v, rows2_v = bufs[p][:4]
        sem = bufs[p][6]
        return (pltpu.make_async_copy(emb_hbm.at[idx1_v], rows1_v, sem),
                pltpu.make_async_copy(emb_hbm.at[idx2_v], rows2_v, sem))

    def o_cp(j, p):
        out_v, sem = bufs[p][4], bufs[p][7]
        return pltpu.make_async_copy(out_v, out_hbm.at[pl.ds(off_of(j), _CH)],
                                     sem)

    def compute(p):
        rows1_v, rows2_v, out_v = bufs[p][2], bufs[p][3], bufs[p][4]

        def group_body(g, carry):
            gbase = pl.multiple_of(g * 16, 8)
            res = jnp.zeros((16,), jnp.float32)
            for e in range(16):
                i = gbase + e
                acc = rows1_v[i, pl.ds(0, 16)] * rows2_v[i, pl.ds(0, 16)]
                for c in range(1, _EMB_DIM // 16):
                    acc = acc + (rows1_v[i, pl.ds(c * 16, 16)]
                                 * rows2_v[i, pl.ds(c * 16, 16)])
                for k in (1, 2, 4, 8):
                    perm = jnp.bitwise_xor(lanes, k)
                    acc = acc + acc.at[perm].get(mode="promise_in_bounds")
                res = jnp.where(lanes == e, acc, res)
            out_v[pl.ds(gbase, 16)] = 1.0 - jnp.exp(-(res + _EPS))
            return carry

        lax.fori_loop(0, _CH // 16, group_body, 0)

    def iteration(j, p, idx_pref, g_pref, o_wait):
        q = 1 - p
        for cp in g_cps(p):
            cp.wait()
        if idx_pref:
            for cp in idx_cps(j + 2, p):
                cp.start()
        if g_pref:
            for cp in idx_cps(j + 1, q):
                cp.wait()
            for cp in g_cps(q):
                cp.start()
        if o_wait:
            o_cp(j - 2, p).wait()
        compute(p)
        o_cp(j, p).start()

    for cp in idx_cps(0, 0):
        cp.start()
    for cp in idx_cps(1, 1):
        cp.start()
    for cp in idx_cps(0, 0):
        cp.wait()
    for cp in g_cps(0):
        cp.start()

    iteration(0, 0, True, True, False)
    iteration(1, 1, True, True, False)

    def pair_body(jj, carry):
        j0 = jj * 2
        iteration(j0, 0, True, True, True)
        iteration(j0 + 1, 1, True, True, True)
        return carry

    lax.fori_loop(1, (_NCH - 3) // 2, pair_body, 0)

    iteration(_NCH - 3, 0, True, True, True)
    iteration(_NCH - 2, 1, False, True, True)
    iteration(_NCH - 1, 0, False, False, True)
    o_cp(_NCH - 2, 1).wait()
    o_cp(_NCH - 1, 0).wait()


def kernel(emb, idx):
    e1 = idx[:, 0]
    e2 = idx[:, 1]
    return _berpo_sc(emb, e1, e2)


# speedup vs baseline: 1.0149x; 1.0149x over previous
import functools

import jax
import jax.numpy as jnp
import numpy as np
from jax import lax
from jax.experimental import pallas as pl
from jax.experimental.pallas import tpu as pltpu
from jax.experimental.pallas import tpu_sc as plsc

_NUM_NODES = 10000
_NUM_EDGES = 320000
_EMB_DIM = 128
_EDGE_PROBA = _NUM_EDGES / (_NUM_NODES ** 2 - _NUM_NODES)
_EPS = np.float32(-np.log(1.0 - _EDGE_PROBA))

_NW = 32
_E_PER_W = _NUM_EDGES // _NW
_CH = 80
_NCH = _E_PER_W // _CH
_NB = 3

_mesh = plsc.VectorSubcoreMesh(core_axis_name="c", subcore_axis_name="s")


@functools.partial(
    pl.kernel,
    mesh=_mesh,
    out_type=jax.ShapeDtypeStruct((_NUM_EDGES,), jnp.float32),
    scratch_types=(
        [pltpu.VMEM((_CH,), jnp.int32)] * (2 * _NB)
        + [pltpu.VMEM((_CH, _EMB_DIM), jnp.float32)] * (2 * _NB)
        + [pltpu.VMEM((_CH,), jnp.float32)] * _NB
        + [pltpu.SemaphoreType.DMA] * (3 * _NB)
    ),
)
def _berpo_sc(emb_hbm, e1_hbm, e2_hbm, out_hbm,
              i1_0, i2_0, i1_1, i2_1, i1_2, i2_2,
              r1_0, r2_0, r1_1, r2_1, r1_2, r2_2,
              o_0, o_1, o_2,
              si_0, si_1, si_2, sg_0, sg_1, sg_2, so_0, so_1, so_2):
    wid = lax.axis_index("s") * 2 + lax.axis_index("c")
    base = wid * _E_PER_W
    lanes = lax.iota(jnp.int32, 16)

    idx1_v = (i1_0, i1_1, i1_2)
    idx2_v = (i2_0, i2_1, i2_2)
    rows1_v = (r1_0, r1_1, r1_2)
    rows2_v = (r2_0, r2_1, r2_2)
    out_v = (o_0, o_1, o_2)
    sem_i = (si_0, si_1, si_2)
    sem_g = (sg_0, sg_1, sg_2)
    sem_o = (so_0, so_1, so_2)

    def off_of(j):
        return pl.multiple_of(base + j * _CH, 8)

    def idx_cps(j, p):
        off = off_of(j)
        return (
            pltpu.make_async_copy(e1_hbm.at[pl.ds(off, _CH)], idx1_v[p],
                                  sem_i[p]),
            pltpu.make_async_copy(e2_hbm.at[pl.ds(off, _CH)], idx2_v[p],
                                  sem_i[p]),
        )

    def g_cps(p):
        return (
            pltpu.make_async_copy(emb_hbm.at[idx1_v[p]], rows1_v[p],
                                  sem_g[p]),
            pltpu.make_async_copy(emb_hbm.at[idx2_v[p]], rows2_v[p],
                                  sem_g[p]),
        )

    def o_cp(j, p):
        return pltpu.make_async_copy(
            out_v[p], out_hbm.at[pl.ds(off_of(j), _CH)], sem_o[p])

    def edge_result(r1, r2, i):
        acc = r1[i, pl.ds(0, 16)] * r2[i, pl.ds(0, 16)]
        for c in range(1, _EMB_DIM // 16):
            acc = acc + (r1[i, pl.ds(c * 16, 16)]
                         * r2[i, pl.ds(c * 16, 16)])
        for k in (1, 2, 4, 8):
            perm = jnp.bitwise_xor(lanes, k)
            acc = acc + acc.at[perm].get(mode="promise_in_bounds")
        return acc

    def compute(p):
        r1, r2, o = rows1_v[p], rows2_v[p], out_v[p]

        def group_body(g, carry):
            gbase = pl.multiple_of(g * 16, 8)
            res = jnp.zeros((16,), jnp.float32)
            for e in range(16):
                acc = edge_result(r1, r2, gbase + e)
                res = jnp.where(lanes == e, acc, res)
            o[pl.ds(gbase, 16)] = 1.0 - jnp.exp(-(res + _EPS))
            return carry

        lax.fori_loop(0, _CH // 16, group_body, 0)

    def compute_lite(p):
        r1, r2, o = rows1_v[p], rows2_v[p], out_v[p]

        def group_body(g, carry):
            gbase = pl.multiple_of(g * 16, 8)

            def edge_body(e, res):
                acc = edge_result(r1, r2, gbase + e)
                return jnp.where(lanes == e, acc, res)

            res = lax.fori_loop(0, 16, edge_body,
                                jnp.zeros((16,), jnp.float32))
            o[pl.ds(gbase, 16)] = 1.0 - jnp.exp(-(res + _EPS))
            return carry

        lax.fori_loop(0, _CH // 16, group_body, 0)

    def iteration(j, p, i_pref, g_pref, o_wait, lite):
        for cp in g_cps(p):
            cp.wait()
        if g_pref:
            q = (p + 2) % _NB
            for cp in idx_cps(j + 2, q):
                cp.wait()
            for cp in g_cps(q):
                cp.start()
        if i_pref:
            for cp in idx_cps(j + _NB, p):
                cp.start()
        if o_wait:
            o_cp(j - _NB, p).wait()
        (compute_lite if lite else compute)(p)
        o_cp(j, p).start()

    for p in range(_NB):
        for cp in idx_cps(p, p):
            cp.start()
    for p in range(2):
        for cp in idx_cps(p, p):
            cp.wait()
        for cp in g_cps(p):
            cp.start()

    iteration(0, 0, True, True, False, True)
    iteration(1, 1, True, True, False, True)
    iteration(2, 2, True, True, False, True)

    def triple_body(jj, carry):
        j0 = jj * _NB
        iteration(j0, 0, True, True, True, False)
        iteration(j0 + 1, 1, True, True, True, False)
        iteration(j0 + 2, 2, True, True, True, False)
        return carry

    lax.fori_loop(1, (_NCH - 5) // _NB, triple_body, 0)

    iteration(_NCH - 5, 0, True, True, True, True)
    iteration(_NCH - 4, 1, True, True, True, True)
    iteration(_NCH - 3, 2, False, True, True, True)
    iteration(_NCH - 2, 0, False, False, True, True)
    iteration(_NCH - 1, 1, False, False, True, True)
    o_cp(_NCH - 3, 2).wait()
    o_cp(_NCH - 2, 0).wait()
    o_cp(_NCH - 1, 1).wait()


def kernel(emb, idx):
    e1 = idx[:, 0]
    e2 = idx[:, 1]
    return _berpo_sc(emb, e1, e2)
